# baseline (speedup 1.0000x reference)
import jax
import jax.numpy as jnp
from jax.experimental import pallas as pl

B = 16384
N = B * 64


def _body(gu_ref, gi_ref, out_ref):
    out_ref[...] = gu_ref[pl.ds(0, 2048)] + gi_ref[pl.ds(0, 2048)]


def kernel(gu, gi, bu, bi, Mu):
    out = pl.pallas_call(
        _body,
        grid=(8,),
        in_specs=[
            pl.BlockSpec((N // 8,), lambda i: (i,)),
            pl.BlockSpec((N // 8,), lambda i: (i,)),
        ],
        out_specs=pl.BlockSpec((2048,), lambda i: (i,)),
        out_shape=jax.ShapeDtypeStruct((B,), jnp.float32),
    )(gu.reshape(N), gi.reshape(N))
    return out


# speedup vs baseline: 4.4787x; 4.4787x over previous
import jax
import jax.numpy as jnp
from jax.experimental import pallas as pl

B = 16384
K = 64
BLKC = 2048


def _body(gu_ref, gi_ref, bu_ref, bi_ref, mu_ref, out_ref):
    prod = gu_ref[...] * gi_ref[...]
    s = jnp.sum(prod, axis=0, keepdims=True)
    out_ref[...] = s + bu_ref[...] + bi_ref[...] + mu_ref[0, 0]


def kernel(gu, gi, bu, bi, Mu):
    gut = gu.T
    git = gi.T
    but = bu.T
    bit = bi.T
    grid = (B // BLKC,)
    out = pl.pallas_call(
        _body,
        grid=grid,
        in_specs=[
            pl.BlockSpec((K, BLKC), lambda i: (0, i)),
            pl.BlockSpec((K, BLKC), lambda i: (0, i)),
            pl.BlockSpec((1, BLKC), lambda i: (0, i)),
            pl.BlockSpec((1, BLKC), lambda i: (0, i)),
            pl.BlockSpec((1, 1), lambda i: (0, 0)),
        ],
        out_specs=pl.BlockSpec((1, BLKC), lambda i: (0, i)),
        out_shape=jax.ShapeDtypeStruct((1, B), jnp.float32),
    )(gut, git, but, bit, Mu)
    return out.reshape(B)
